# Initial kernel scaffold; baseline (speedup 1.0000x reference)
import jax
import jax.numpy as jnp
from jax import lax
from jax.experimental import pallas as pl
from jax.experimental.pallas import tpu as pltpu
from jax.experimental.pallas import tpu_sc as plsc

EMBED_DIM = 64
NUM_PACKETS = 130560
CHUNK = 510
SEQ = CHUNK + 2
NUM_CHUNKS = NUM_PACKETS // CHUNK

_NC = 2
_NS = 16
_NW = _NC * _NS
_CPW = NUM_CHUNKS // _NW
_LANES = 16
_NT = EMBED_DIM // _LANES


def _body(x_hbm, dir_hbm, tok_hbm, dtab_hbm, pos_hbm, out_hbm,
          out_buf, pos_v, tok_v, dtab_v, d_v):
    wid = lax.axis_index("c") * _NS + lax.axis_index("s")

    pltpu.sync_copy(tok_hbm, tok_v)
    pltpu.sync_copy(dtab_hbm, dtab_v)
    pltpu.sync_copy(pos_hbm, pos_v)

    t0 = [dtab_v[0, pl.ds(_LANES * t, _LANES)] for t in range(_NT)]
    t1 = [dtab_v[1, pl.ds(_LANES * t, _LANES)] for t in range(_NT)]
    t2 = [dtab_v[2, pl.ds(_LANES * t, _LANES)] for t in range(_NT)]

    for t in range(_NT):
        sl = pl.ds(_LANES * t, _LANES)
        out_buf[0, sl] = tok_v[1, sl] * 2.0 + pos_v[0, sl]
        out_buf[SEQ - 1, sl] = tok_v[2, sl] * 2.0 + pos_v[SEQ - 1, sl]

    def do_chunk(k, carry):
        cc = wid * _CPW + k
        pltpu.sync_copy(
            x_hbm.at[pl.ds(cc * CHUNK, CHUNK), pl.ds(0, EMBED_DIM)],
            out_buf.at[pl.ds(1, CHUNK)],
        )
        pltpu.sync_copy(dir_hbm.at[pl.ds(cc, 1)], d_v)

        def row(j, c2):
            d = d_v[0, j]
            is0 = d == 0
            is1 = d == 1
            r = j + 1
            for t in range(_NT):
                sl = pl.ds(_LANES * t, _LANES)
                sel = jnp.where(is0, t0[t], jnp.where(is1, t1[t], t2[t]))
                out_buf[r, sl] = out_buf[r, sl] + pos_v[r, sl] + sel
            return c2

        lax.fori_loop(0, CHUNK, row, 0)
        pltpu.sync_copy(out_buf, out_hbm.at[cc])
        return carry

    lax.fori_loop(0, _CPW, do_chunk, 0)


@jax.jit
def _flow_embed(x2, dir2, tok, dtab, pos):
    mesh = plsc.VectorSubcoreMesh(core_axis_name="c", subcore_axis_name="s")
    f = pl.kernel(
        _body,
        out_type=jax.ShapeDtypeStruct((NUM_CHUNKS, SEQ, EMBED_DIM), jnp.float32),
        mesh=mesh,
        scratch_types=[
            pltpu.VMEM((SEQ, EMBED_DIM), jnp.float32),
            pltpu.VMEM((SEQ, EMBED_DIM), jnp.float32),
            pltpu.VMEM((5, EMBED_DIM), jnp.float32),
            pltpu.VMEM((3, EMBED_DIM), jnp.float32),
            pltpu.VMEM((1, CHUNK), jnp.int32),
        ],
    )
    return f(x2, dir2, tok, dtab, pos)


def kernel(cls_packet_embeddings, direction, token_embed, direction_embed,
           packet_pos_embed):
    x2 = cls_packet_embeddings.reshape(NUM_PACKETS, 2 * EMBED_DIM)
    dir2 = direction.astype(jnp.int32).reshape(NUM_CHUNKS, CHUNK)
    embed_val = _flow_embed(x2, dir2, token_embed, direction_embed,
                            packet_pos_embed)
    pad_indices = jnp.zeros((NUM_CHUNKS, SEQ), dtype=bool)
    pad_indices = pad_indices.at[:, 0].set(True).at[:, -1].set(True)
    return (embed_val, pad_indices)


# speedup vs baseline: 1.8730x; 1.8730x over previous
import jax
import jax.numpy as jnp
from jax import lax
from jax.experimental import pallas as pl
from jax.experimental.pallas import tpu as pltpu
from jax.experimental.pallas import tpu_sc as plsc

EMBED_DIM = 64
NUM_PACKETS = 130560
CHUNK = 510
SEQ = CHUNK + 2
NUM_CHUNKS = NUM_PACKETS // CHUNK

_NC = 2
_NS = 16
_NW = _NC * _NS
_CPW = NUM_CHUNKS // _NW
_LANES = 16
_NT = EMBED_DIM // _LANES


def _body(x_hbm, dir_hbm, tok_hbm, dtab_hbm, pos_hbm, out_hbm,
          out_buf, pos_v, tok_v, dtab_v, d_v):
    wid = lax.axis_index("c") * _NS + lax.axis_index("s")

    pltpu.sync_copy(tok_hbm, tok_v)
    pltpu.sync_copy(dtab_hbm, dtab_v)
    pltpu.sync_copy(pos_hbm, pos_v)

    t0 = [dtab_v[0, pl.ds(_LANES * t, _LANES)] for t in range(_NT)]
    t1 = [dtab_v[1, pl.ds(_LANES * t, _LANES)] for t in range(_NT)]
    t2 = [dtab_v[2, pl.ds(_LANES * t, _LANES)] for t in range(_NT)]

    for t in range(_NT):
        sl = pl.ds(_LANES * t, _LANES)
        out_buf[0, sl] = tok_v[1, sl] * 2.0 + pos_v[0, sl]
        out_buf[SEQ - 1, sl] = tok_v[2, sl] * 2.0 + pos_v[SEQ - 1, sl]

    def do_chunk(k, carry):
        cc = wid * _CPW + k
        pltpu.sync_copy(
            x_hbm.at[pl.ds(cc * CHUNK, CHUNK), pl.ds(0, EMBED_DIM)],
            out_buf.at[pl.ds(1, CHUNK)],
        )
        pltpu.sync_copy(dir_hbm.at[pl.ds(cc, 1)], d_v)

        def rowgroup(g, c2):
            d16 = d_v[0, pl.ds(g * _LANES, _LANES)]
            base = g * _LANES + 1
            for jj in range(_LANES):
                d = d16[jj]
                is0 = d == 0
                is1 = d == 1
                r = base + jj
                for t in range(_NT):
                    sl = pl.ds(_LANES * t, _LANES)
                    sel = jnp.where(is0, t0[t], jnp.where(is1, t1[t], t2[t]))
                    out_buf[r, sl] = out_buf[r, sl] + pos_v[r, sl] + sel
            return c2

        lax.fori_loop(0, CHUNK // _LANES, rowgroup, 0)
        dtail = d_v[0, pl.ds(CHUNK - _LANES, _LANES)]
        for jj in range(2, _LANES):
            d = dtail[jj]
            is0 = d == 0
            is1 = d == 1
            r = (CHUNK - _LANES) + jj + 1
            for t in range(_NT):
                sl = pl.ds(_LANES * t, _LANES)
                sel = jnp.where(is0, t0[t], jnp.where(is1, t1[t], t2[t]))
                out_buf[r, sl] = out_buf[r, sl] + pos_v[r, sl] + sel
        pltpu.sync_copy(out_buf, out_hbm.at[cc])
        return carry

    lax.fori_loop(0, _CPW, do_chunk, 0)


@jax.jit
def _flow_embed(x2, dir2, tok, dtab, pos):
    mesh = plsc.VectorSubcoreMesh(core_axis_name="c", subcore_axis_name="s")
    f = pl.kernel(
        _body,
        out_type=jax.ShapeDtypeStruct((NUM_CHUNKS, SEQ, EMBED_DIM), jnp.float32),
        mesh=mesh,
        scratch_types=[
            pltpu.VMEM((SEQ, EMBED_DIM), jnp.float32),
            pltpu.VMEM((SEQ, EMBED_DIM), jnp.float32),
            pltpu.VMEM((5, EMBED_DIM), jnp.float32),
            pltpu.VMEM((3, EMBED_DIM), jnp.float32),
            pltpu.VMEM((1, CHUNK), jnp.int32),
        ],
        compiler_params=pltpu.CompilerParams(use_tc_tiling_on_sc=False),
    )
    return f(x2, dir2, tok, dtab, pos)


def kernel(cls_packet_embeddings, direction, token_embed, direction_embed,
           packet_pos_embed):
    x2 = cls_packet_embeddings.reshape(NUM_PACKETS, 2 * EMBED_DIM)
    dir2 = direction.astype(jnp.int32).reshape(NUM_CHUNKS, CHUNK)
    embed_val = _flow_embed(x2, dir2, token_embed, direction_embed,
                            packet_pos_embed)
    pad_indices = jnp.zeros((NUM_CHUNKS, SEQ), dtype=bool)
    pad_indices = pad_indices.at[:, 0].set(True).at[:, -1].set(True)
    return (embed_val, pad_indices)
